# Initial kernel scaffold; baseline (speedup 1.0000x reference)
import jax
import jax.numpy as jnp
from jax.experimental import pallas as pl


def kernel(inputs, edge_index_bwd, edge_index_fwd, W_gcn, b_gcn, W_conv, b_conv, W_lin, b_lin):
    raise NotImplementedError("write your pallas kernel here")


# speedup vs baseline: 137.5863x; 137.5863x over previous
import functools

import jax
import jax.numpy as jnp
from jax import lax
from jax.experimental import pallas as pl
from jax.experimental.pallas import tpu as pltpu
from jax.experimental.pallas import tpu_sc as plsc

_B, _T, _N, _D = 4, 8, 1024, 128
_C, _L = 2, 2
_E = 32768
_P = 4


def _build_adjacency(edge_bwd, edge_fwd):
    info = plsc.get_sparse_core_info()
    n_sub = info.num_subcores
    lanes = info.num_lanes
    e_per_tile = _E // n_sub
    rows_per_tile = (_N * _N) // n_sub

    zeros_hbm = jnp.zeros((rows_per_tile,), jnp.float32)
    ones_hbm = jnp.ones((e_per_tile,), jnp.float32)

    mesh = plsc.VectorSubcoreMesh(core_axis_name="c", subcore_axis_name="s")

    @functools.partial(
        pl.kernel,
        mesh=mesh,
        out_type=jax.ShapeDtypeStruct((2, _N * _N), jnp.float32),
        scratch_types=[
            pltpu.VMEM((e_per_tile,), jnp.int32),
            pltpu.VMEM((e_per_tile,), jnp.int32),
            pltpu.VMEM((e_per_tile,), jnp.int32),
            pltpu.VMEM((e_per_tile,), jnp.float32),
            pltpu.VMEM_SHARED((_N * _N,), jnp.float32),
        ],
    )
    def build(e_bwd, e_fwd, zeros_in, ones_in, a_out, src_v, dst_v, idx_v,
              ones_v, a_sh):
        cid = lax.axis_index("c")
        sid = lax.axis_index("s")
        row0 = sid * rows_per_tile
        ebase = sid * e_per_tile

        pltpu.sync_copy(zeros_in, a_sh.at[pl.ds(row0, rows_per_tile)])
        pltpu.sync_copy(ones_in, ones_v)

        def scatter_edges(e_ref):
            pltpu.sync_copy(e_ref.at[0, pl.ds(ebase, e_per_tile)], src_v)
            pltpu.sync_copy(e_ref.at[1, pl.ds(ebase, e_per_tile)], dst_v)

            def body(i, _):
                sl = pl.ds(i * lanes, lanes)
                idx_v[sl] = dst_v[sl] * _N + src_v[sl]
                return ()

            lax.fori_loop(0, e_per_tile // lanes, body, ())
            plsc.subcore_barrier()
            pltpu.sync_copy(ones_v, a_sh.at[idx_v], add=True)
            plsc.subcore_barrier()

        @pl.when(cid == 0)
        def _():
            scatter_edges(e_bwd)
            pltpu.sync_copy(a_sh.at[pl.ds(row0, rows_per_tile)],
                            a_out.at[0, pl.ds(row0, rows_per_tile)])

        @pl.when(cid == 1)
        def _():
            scatter_edges(e_fwd)
            pltpu.sync_copy(a_sh.at[pl.ds(row0, rows_per_tile)],
                            a_out.at[1, pl.ds(row0, rows_per_tile)])

    return build(edge_bwd, edge_fwd, zeros_hbm, ones_hbm).reshape(2, _N, _N)


def _tc_body(x_ref, a_ref, wg_ref, bg_ref, wc_ref, wl_ref, ob_ref, out_ref):
    acc = None
    for half in range(2):
        a = a_ref[half]
        deg = jnp.maximum(jnp.sum(a, axis=1, keepdims=True), 1.0)
        inv = 1.0 / deg
        h_in = x_ref[0, half]
        m0 = jnp.dot(a, h_in, preferred_element_type=jnp.float32)
        g0 = m0 * inv + h_in
        z = None
        for c in range(_C):
            h1 = jnp.maximum(
                jnp.dot(g0, wg_ref[c, half, 0],
                        preferred_element_type=jnp.float32) + bg_ref[c, half, 0],
                0.0)
            m1 = jnp.dot(a, h1, preferred_element_type=jnp.float32)
            g1 = m1 * inv + h1
            h2 = jnp.maximum(
                jnp.dot(g1, wg_ref[c, half, 1],
                        preferred_element_type=jnp.float32) + bg_ref[c, half, 1],
                0.0)
            zc = jnp.dot(h2, wl_ref[c], preferred_element_type=jnp.float32)
            z = zc if z is None else z + zc
        y = jnp.dot(wc_ref[:, half * _N:(half + 1) * _N], z,
                    preferred_element_type=jnp.float32)
        acc = y if acc is None else acc + y
    out_ref[0, 0] = acc + ob_ref[...]


def kernel(inputs, edge_index_bwd, edge_index_fwd, W_gcn, b_gcn, W_conv,
           b_conv, W_lin, b_lin):
    a2 = _build_adjacency(edge_index_bwd, edge_index_fwd)

    bg = b_gcn.reshape(_C, 2, _L, 1, _D)
    wl = W_lin.reshape(_C, _D, _D)
    out_bias = b_conv[:, None] * jnp.sum(W_lin, axis=0)[None, :] + b_lin[None, :]

    grid = (_B, _P)
    out = pl.pallas_call(
        _tc_body,
        grid=grid,
        in_specs=[
            pl.BlockSpec((1, 2, _N, _D), lambda b, p: (b, p, 0, 0)),
            pl.BlockSpec((2, _N, _N), lambda b, p: (0, 0, 0)),
            pl.BlockSpec((_C, 2, _L, _D, _D), lambda b, p: (0, 0, 0, 0, 0)),
            pl.BlockSpec((_C, 2, _L, 1, _D), lambda b, p: (0, 0, 0, 0, 0)),
            pl.BlockSpec((_N, 2 * _N), lambda b, p: (0, 0)),
            pl.BlockSpec((_C, _D, _D), lambda b, p: (0, 0, 0)),
            pl.BlockSpec((_N, _D), lambda b, p: (0, 0)),
        ],
        out_specs=pl.BlockSpec((1, 1, _N, _D), lambda b, p: (b, p, 0, 0)),
        out_shape=jax.ShapeDtypeStruct((_B, _P, _N, _D), jnp.float32),
        compiler_params=pltpu.CompilerParams(
            dimension_semantics=("parallel", "parallel"),
            vmem_limit_bytes=100 * 1024 * 1024,
        ),
    )(inputs, a2, W_gcn, bg, W_conv, wl, out_bias)
    return out
